# baseline (speedup 1.0000x reference)
import functools

import jax
import jax.numpy as jnp
from jax import lax
from jax.experimental import pallas as pl
from jax.experimental.pallas import tpu as pltpu
from jax.experimental.pallas import tpu_sc as plsc

_NUM_CLASSES = 16
_NC = 2
_NS = 16
_NW = _NC * _NS
_N = 4096
_F = 8192
_FPT = _F // _NW
_BATCH = 64
_NBATCH = _N // _BATCH


def _sc_segment_sum(x2d, labels):
    mesh = plsc.VectorSubcoreMesh(core_axis_name="c", subcore_axis_name="s")

    @functools.partial(
        pl.kernel,
        out_type=jax.ShapeDtypeStruct((_NUM_CLASSES, _F), jnp.float32),
        mesh=mesh,
        scratch_types=[
            pltpu.VMEM((_N,), jnp.int32),
            pltpu.VMEM((_BATCH, _FPT), jnp.float32),
            pltpu.VMEM((_BATCH, _FPT), jnp.float32),
            pltpu.VMEM((_NUM_CLASSES, _FPT), jnp.float32),
            pltpu.SemaphoreType.DMA,
            pltpu.SemaphoreType.DMA,
        ],
        compiler_params=pltpu.CompilerParams(
            use_tc_tiling_on_sc=False, needs_layout_passes=False
        ),
    )
    def seg_sum(x_hbm, lab_hbm, out_hbm, lab_v, buf_a, buf_b, acc_v, sem_a, sem_b):
        c = lax.axis_index("c")
        s = lax.axis_index("s")
        wid = c * _NS + s
        col0 = wid * _FPT

        pltpu.sync_copy(lab_hbm, lab_v)

        zeros = jnp.zeros((16,), jnp.float32)

        def _zero_body(i, _):
            r = i // (_FPT // 16)
            k = i % (_FPT // 16)
            acc_v[r, pl.ds(k * 16, 16)] = zeros
            return 0

        lax.fori_loop(0, _NUM_CLASSES * (_FPT // 16), _zero_body, 0)

        col_iota = lax.broadcasted_iota(jnp.int32, (16,), 0)
        _gdn = lax.GatherDimensionNumbers(
            offset_dims=(), collapsed_slice_dims=(0,), start_index_map=(0,)
        )

        def _splat(v, j):
            idx = jnp.full((16, 1), j, jnp.int32)
            return lax.gather(
                v, idx, _gdn, (1,),
                mode=lax.GatherScatterMode.PROMISE_IN_BOUNDS,
            )

        def _start(j, buf, sem):
            pltpu.async_copy(
                x_hbm.at[pl.ds(j * _BATCH, _BATCH), pl.ds(col0, _FPT)],
                buf,
                sem,
            )

        def _wait(buf, sem):
            pltpu.make_async_copy(
                x_hbm.at[pl.ds(0, _BATCH), pl.ds(col0, _FPT)], buf, sem
            ).wait()

        def _process(j, buf):
            n0 = j * _BATCH

            def _group_body(g, _):
                labv = lab_v[pl.ds(n0 + g * 16, 16)]
                for jj in range(16):
                    row_idx = _splat(labv, jj)
                    for k in range(_FPT // 16):
                        vals = buf[g * 16 + jj, pl.ds(k * 16, 16)]
                        plsc.addupdate_scatter(
                            acc_v, [row_idx, col_iota + (k * 16)], vals
                        )
                return 0

            lax.fori_loop(0, _BATCH // 16, _group_body, 0)

        _start(0, buf_a, sem_a)
        _start(1, buf_b, sem_b)

        def _pipe_body(h, _):
            j0 = h * 2

            _wait(buf_a, sem_a)
            _process(j0, buf_a)

            @pl.when(j0 + 2 < _NBATCH)
            def _():
                _start(j0 + 2, buf_a, sem_a)

            _wait(buf_b, sem_b)
            _process(j0 + 1, buf_b)

            @pl.when(j0 + 3 < _NBATCH)
            def _():
                _start(j0 + 3, buf_b, sem_b)

            return 0

        lax.fori_loop(0, _NBATCH // 2, _pipe_body, 0)

        pltpu.sync_copy(acc_v, out_hbm.at[:, pl.ds(col0, _FPT)])

    return seg_sum(x2d, labels)


def _tc_body(sums_ref, lab_ref, out_ref):
    sums = sums_ref[...]
    lab = lab_ref[...]
    onehot = (
        lab[None, :, :]
        == lax.broadcasted_iota(jnp.int32, (_NUM_CLASSES, 32, _N // 32), 0)
    ).astype(jnp.float32)
    counts = jnp.sum(onehot, axis=(1, 2))
    denom = jnp.maximum(counts, 1.0)[:, None]
    centers = jnp.where(counts[:, None] > 0, sums / denom, 0.0)

    rows = []
    for i in range(_NUM_CLASSES):
        diff = centers - centers[i][None, :]
        rows.append(jnp.sum(diff * diff, axis=1))
    sq = jnp.stack(rows)

    ii = lax.broadcasted_iota(jnp.int32, (_NUM_CLASSES, _NUM_CLASSES), 0)
    jj = lax.broadcasted_iota(jnp.int32, (_NUM_CLASSES, _NUM_CLASSES), 1)
    mask = jj > ii
    norms = jnp.where(mask, jnp.sqrt(jnp.where(mask, sq, 1.0)), 0.0)
    num = _NUM_CLASSES * (_NUM_CLASSES - 1) // 2
    out_ref[...] = (jnp.sum(norms) / num).reshape(1, 1)


def _tc_distance(sums, labels2d):
    return pl.pallas_call(
        _tc_body,
        out_shape=jax.ShapeDtypeStruct((1, 1), jnp.float32),
    )(sums, labels2d)


@jax.jit
def kernel(x, labels):
    if x.ndim == 4:
        n, c, h, w = x.shape
        x = x.reshape(n, c * h, w)
    x2d = x.reshape(_N, _F)
    labels = labels.astype(jnp.int32)
    sums = _sc_segment_sum(x2d, labels)
    labels2d = labels.reshape(32, _N // 32)
    out = _tc_distance(sums, labels2d)
    return out[0, 0]
